# Initial kernel scaffold; baseline (speedup 1.0000x reference)
import jax
import jax.numpy as jnp
from jax.experimental import pallas as pl


def kernel(features, edge_index, edge_attr, W1, b1, W2, b2):
    raise NotImplementedError("write your pallas kernel here")


# speedup vs baseline: 12.8681x; 12.8681x over previous
import functools

import jax
import jax.numpy as jnp
from jax import lax
from jax.experimental import pallas as pl
from jax.experimental.pallas import tpu as pltpu
from jax.experimental.pallas import tpu_sc as plsc

NC = 2
NS = 16
NW = NC * NS

_K = 80


def _deg_partials(dst, ones_k, zeros_n, n):
    e = dst.shape[0]
    ec = e // NW
    nchunk = ec // _K
    mesh = plsc.VectorSubcoreMesh(core_axis_name="c", subcore_axis_name="s")

    @functools.partial(
        pl.kernel,
        out_type=[jax.ShapeDtypeStruct((n,), jnp.float32)] * NC,
        mesh=mesh,
        scratch_types=[
            pltpu.VMEM((_K,), jnp.int32),
            pltpu.VMEM((_K,), jnp.float32),
            pltpu.VMEM_SHARED((n,), jnp.float32),
        ],
    )
    def body(dst_hbm, ones_hbm, zeros_hbm, out0_hbm, out1_hbm,
             didx, ones_v, accum):
        c = lax.axis_index("c")
        s = lax.axis_index("s")
        wid = s * NC + c
        pltpu.sync_copy(ones_hbm, ones_v)

        @pl.when(s == 0)
        def _init():
            pltpu.sync_copy(zeros_hbm, accum)

        plsc.subcore_barrier()
        base = wid * ec

        @pl.loop(0, nchunk)
        def _chunk(i):
            off = base + i * _K
            pltpu.sync_copy(dst_hbm.at[pl.ds(off, _K)], didx)
            pltpu.sync_copy(ones_v, accum.at[didx], add=True)

        plsc.subcore_barrier()

        @pl.when((s == 0) & (c == 0))
        def _flush0():
            pltpu.sync_copy(accum, out0_hbm)

        @pl.when((s == 0) & (c == 1))
        def _flush1():
            pltpu.sync_copy(accum, out1_hbm)

    return body(dst, ones_k, zeros_n)


def _agg_partials(hp, src, dst, zeros_nd):
    n, d = hp.shape
    e = src.shape[0]
    ec = e // NW
    nchunk = ec // _K
    rpt = n // NS
    mesh = plsc.VectorSubcoreMesh(core_axis_name="c", subcore_axis_name="s")

    @functools.partial(
        pl.kernel,
        out_type=jax.ShapeDtypeStruct((NC, n, d), jnp.float32),
        mesh=mesh,
        scratch_types=[
            pltpu.VMEM((_K,), jnp.int32),
            pltpu.VMEM((_K,), jnp.int32),
            pltpu.VMEM((_K, d), jnp.float32),
            pltpu.VMEM_SHARED((n, d), jnp.float32),
            pltpu.SemaphoreType.DMA,
        ],
    )
    def body(hp_hbm, src_hbm, dst_hbm, zeros_hbm, out_hbm,
             sidx, didx, rows, accum, sem):
        c = lax.axis_index("c")
        s = lax.axis_index("s")
        wid = s * NC + c

        @pl.when(s == 0)
        def _init():
            pltpu.sync_copy(zeros_hbm, accum)

        plsc.subcore_barrier()
        base = wid * ec

        @pl.loop(0, nchunk)
        def _chunk(i):
            off = base + i * _K
            pltpu.sync_copy(src_hbm.at[pl.ds(off, _K)], sidx)
            pltpu.async_copy(hp_hbm.at[sidx], rows, sem).wait()
            pltpu.sync_copy(dst_hbm.at[pl.ds(off, _K)], didx)
            pltpu.sync_copy(rows, accum.at[didx], add=True)

        plsc.subcore_barrier()

        @pl.when(s == 0)
        def _flush():
            pltpu.sync_copy(accum, out_hbm.at[c])

    return body(hp, src, dst, zeros_nd)


def _dis_from(d0_ref, d1_ref):
    deg = 1.0 + d0_ref[...] + d1_ref[...]
    return lax.rsqrt(deg)


_BN = 1000


def _scaled_matmul(degp, x, w):
    n, d = x.shape
    h = w.shape[1]

    def body(d0_ref, d1_ref, x_ref, w_ref, o_ref):
        dis = _dis_from(d0_ref, d1_ref)
        hm = jnp.dot(x_ref[...], w_ref[...],
                     preferred_element_type=jnp.float32,
                     precision=lax.Precision.HIGHEST)
        o_ref[...] = hm * dis

    return pl.pallas_call(
        body,
        grid=(n // _BN,),
        in_specs=[
            pl.BlockSpec((_BN, 1), lambda i: (i, 0)),
            pl.BlockSpec((_BN, 1), lambda i: (i, 0)),
            pl.BlockSpec((_BN, d), lambda i: (i, 0)),
            pl.BlockSpec((d, h), lambda i: (0, 0)),
        ],
        out_specs=pl.BlockSpec((_BN, h), lambda i: (i, 0)),
        out_shape=jax.ShapeDtypeStruct((n, h), jnp.float32),
    )(degp[0], degp[1], x, w)


def _mid_layer(aggp, hp, degp, b, w):
    n, d = hp.shape
    h = w.shape[1]

    def body(agg_ref, hp_ref, d0_ref, d1_ref, b_ref, w_ref, o_ref):
        dis = _dis_from(d0_ref, d1_ref)
        a = agg_ref[...]
        tot = a[0] + a[1] + hp_ref[...]
        x = jnp.maximum(tot * dis + b_ref[...], 0.0)
        hm = jnp.dot(x, w_ref[...],
                     preferred_element_type=jnp.float32,
                     precision=lax.Precision.HIGHEST)
        o_ref[...] = hm * dis

    return pl.pallas_call(
        body,
        grid=(n // _BN,),
        in_specs=[
            pl.BlockSpec((NC, _BN, d), lambda i: (0, i, 0)),
            pl.BlockSpec((_BN, d), lambda i: (i, 0)),
            pl.BlockSpec((_BN, 1), lambda i: (i, 0)),
            pl.BlockSpec((_BN, 1), lambda i: (i, 0)),
            pl.BlockSpec((1, d), lambda i: (0, 0)),
            pl.BlockSpec((d, h), lambda i: (0, 0)),
        ],
        out_specs=pl.BlockSpec((_BN, h), lambda i: (i, 0)),
        out_shape=jax.ShapeDtypeStruct((n, h), jnp.float32),
    )(aggp, hp, degp[0], degp[1], b, w)


def _final_layer(aggp, hp, degp, b):
    n, d = hp.shape

    def body(agg_ref, hp_ref, d0_ref, d1_ref, b_ref, o_ref):
        dis = _dis_from(d0_ref, d1_ref)
        a = agg_ref[...]
        tot = a[0] + a[1] + hp_ref[...]
        x = jnp.maximum(tot * dis + b_ref[...], 0.0)
        m = jnp.max(x, axis=0, keepdims=True)

        @pl.when(pl.program_id(0) == 0)
        def _init():
            o_ref[...] = jnp.zeros_like(o_ref)

        o_ref[...] = jnp.maximum(o_ref[...], m)

    return pl.pallas_call(
        body,
        grid=(n // _BN,),
        in_specs=[
            pl.BlockSpec((NC, _BN, d), lambda i: (0, i, 0)),
            pl.BlockSpec((_BN, d), lambda i: (i, 0)),
            pl.BlockSpec((_BN, 1), lambda i: (i, 0)),
            pl.BlockSpec((_BN, 1), lambda i: (i, 0)),
            pl.BlockSpec((1, d), lambda i: (0, 0)),
        ],
        out_specs=pl.BlockSpec((1, d), lambda i: (0, 0)),
        out_shape=jax.ShapeDtypeStruct((1, d), jnp.float32),
    )(aggp, hp, degp[0], degp[1], b)


def kernel(features, edge_index, edge_attr, W1, b1, W2, b2):
    n, d = features.shape
    e = edge_index.shape[1]
    assert e % (NW * _K) == 0 and n % NS == 0 and n % _BN == 0
    src = edge_index[0]
    dst = edge_index[1]
    ones_k = jnp.ones((_K,), jnp.float32)
    zeros_n = jnp.zeros((n,), jnp.float32)
    zeros_nd = jnp.zeros((n, d), jnp.float32)
    b1r = b1.reshape(1, -1)
    b2r = b2.reshape(1, -1)

    dp = _deg_partials(dst, ones_k, zeros_n, n)
    degp = (dp[0][:, None], dp[1][:, None])
    h1p = _scaled_matmul(degp, features, W1)
    agg1 = _agg_partials(h1p, src, dst, zeros_nd)
    h2p = _mid_layer(agg1, h1p, degp, b1r, W2)
    agg2 = _agg_partials(h2p, src, dst, zeros_nd)
    return _final_layer(agg2, h2p, degp, b2r)


# speedup vs baseline: 29.4100x; 2.2855x over previous
import functools

import jax
import jax.numpy as jnp
from jax import lax
from jax.experimental import pallas as pl
from jax.experimental.pallas import tpu as pltpu
from jax.experimental.pallas import tpu_sc as plsc

NC = 2
NS = 16
NW = NC * NS

_K = 80


def _deg_partials(dst, ones_k, zeros_n, n):
    e = dst.shape[0]
    ec = e // NW
    nchunk = ec // _K
    mesh = plsc.VectorSubcoreMesh(core_axis_name="c", subcore_axis_name="s")

    @functools.partial(
        pl.kernel,
        out_type=[jax.ShapeDtypeStruct((n,), jnp.float32)] * NC,
        mesh=mesh,
        scratch_types=[
            pltpu.VMEM((e // NW,), jnp.int32),
            pltpu.VMEM((_K,), jnp.int32),
            pltpu.VMEM((_K,), jnp.int32),
            pltpu.VMEM((_K,), jnp.float32),
            pltpu.VMEM_SHARED((n,), jnp.float32),
            pltpu.SemaphoreType.DMA,
            pltpu.SemaphoreType.DMA,
        ],
    )
    def body(dst_hbm, ones_hbm, zeros_hbm, out0_hbm, out1_hbm,
             didx_all, didx0, didx1, ones_v, accum, ssem0, ssem1):
        c = lax.axis_index("c")
        s = lax.axis_index("s")
        wid = s * NC + c
        pltpu.sync_copy(ones_hbm, ones_v)

        @pl.when(s == 0)
        def _init():
            pltpu.sync_copy(zeros_hbm, accum)

        plsc.subcore_barrier()
        base = wid * ec
        pltpu.sync_copy(dst_hbm.at[pl.ds(base, ec)], didx_all)

        def stage_didx(i, didx_b):
            for j in range(_K // 16):
                didx_b[pl.ds(j * 16, 16)] = didx_all[pl.ds(i * _K + j * 16, 16)]

        def do_chunk(i, didx_b, ssem_b):
            @pl.when(i >= 2)
            def _free():
                pltpu.make_async_copy(ones_v, accum.at[didx_b], ssem_b).wait()

            stage_didx(i, didx_b)
            pltpu.async_copy(ones_v, accum.at[didx_b], ssem_b, add=True)

        @pl.loop(0, nchunk)
        def _chunk(i):
            @pl.when(i % 2 == 0)
            def _even():
                do_chunk(i, didx0, ssem0)

            @pl.when(i % 2 == 1)
            def _odd():
                do_chunk(i, didx1, ssem1)

        pltpu.make_async_copy(ones_v, accum.at[didx0], ssem0).wait()
        pltpu.make_async_copy(ones_v, accum.at[didx1], ssem1).wait()
        plsc.subcore_barrier()

        @pl.when((s == 0) & (c == 0))
        def _flush0():
            pltpu.sync_copy(accum, out0_hbm)

        @pl.when((s == 0) & (c == 1))
        def _flush1():
            pltpu.sync_copy(accum, out1_hbm)

    return body(dst, ones_k, zeros_n)


def _agg_partials(hp, src, dst, zeros_nd):
    n, d = hp.shape
    e = src.shape[0]
    ec = e // NW
    nchunk = ec // _K
    rpt = n // NS
    mesh = plsc.VectorSubcoreMesh(core_axis_name="c", subcore_axis_name="s")

    @functools.partial(
        pl.kernel,
        out_type=jax.ShapeDtypeStruct((NC, n, d), jnp.float32),
        mesh=mesh,
        scratch_types=[
            pltpu.VMEM((ec,), jnp.int32),
            pltpu.VMEM((ec,), jnp.int32),
            pltpu.VMEM((_K,), jnp.int32),
            pltpu.VMEM((_K,), jnp.int32),
            pltpu.VMEM((_K, d), jnp.float32),
            pltpu.VMEM((_K, d), jnp.float32),
            pltpu.VMEM_SHARED((n, d), jnp.float32),
            pltpu.SemaphoreType.DMA,
            pltpu.SemaphoreType.DMA,
            pltpu.SemaphoreType.DMA,
            pltpu.SemaphoreType.DMA,
        ],
    )
    def body(hp_hbm, src_hbm, dst_hbm, zeros_hbm, out_hbm,
             sidx_all, didx_all, didx0, didx1, rows0, rows1, accum,
             gsem0, gsem1, ssem0, ssem1):
        c = lax.axis_index("c")
        s = lax.axis_index("s")
        wid = s * NC + c

        @pl.when(s == 0)
        def _init():
            pltpu.sync_copy(zeros_hbm, accum)

        base = wid * ec
        pltpu.sync_copy(src_hbm.at[pl.ds(base, ec)], sidx_all)
        pltpu.sync_copy(dst_hbm.at[pl.ds(base, ec)], didx_all)
        plsc.subcore_barrier()

        def stage_didx(i, didx_b):
            for j in range(_K // 16):
                didx_b[pl.ds(j * 16, 16)] = didx_all[pl.ds(i * _K + j * 16, 16)]

        def issue(i, didx_b, rows_b, gsem_b, ssem_b):
            @pl.when(i >= 2)
            def _free():
                pltpu.make_async_copy(rows_b, accum.at[didx_b], ssem_b).wait()

            stage_didx(i, didx_b)
            pltpu.async_copy(hp_hbm.at[sidx_all.at[pl.ds(i * _K, _K)]],
                             rows_b, gsem_b)

        def complete(i, didx_b, rows_b, gsem_b, ssem_b):
            pltpu.make_async_copy(hp_hbm.at[sidx_all.at[pl.ds(i * _K, _K)]],
                                  rows_b, gsem_b).wait()
            pltpu.async_copy(rows_b, accum.at[didx_b], ssem_b, add=True)

        @pl.loop(0, nchunk + 1)
        def _chunk(i):
            @pl.when((i < nchunk) & (i % 2 == 0))
            def _issue_even():
                issue(i, didx0, rows0, gsem0, ssem0)

            @pl.when((i < nchunk) & (i % 2 == 1))
            def _issue_odd():
                issue(i, didx1, rows1, gsem1, ssem1)

            @pl.when((i >= 1) & (i % 2 == 1))
            def _complete_even():
                complete(i - 1, didx0, rows0, gsem0, ssem0)

            @pl.when((i >= 1) & (i % 2 == 0))
            def _complete_odd():
                complete(i - 1, didx1, rows1, gsem1, ssem1)

        pltpu.make_async_copy(rows0, accum.at[didx0], ssem0).wait()
        pltpu.make_async_copy(rows1, accum.at[didx1], ssem1).wait()
        plsc.subcore_barrier()

        @pl.when(s == 0)
        def _flush():
            pltpu.sync_copy(accum, out_hbm.at[c])

    return body(hp, src, dst, zeros_nd)


def _dis_from(d0_ref, d1_ref):
    deg = 1.0 + d0_ref[...] + d1_ref[...]
    return lax.rsqrt(deg)


_BN = 1000


def _scaled_matmul(degp, x, w):
    n, d = x.shape
    h = w.shape[1]

    def body(d0_ref, d1_ref, x_ref, w_ref, o_ref):
        dis = _dis_from(d0_ref, d1_ref)
        hm = jnp.dot(x_ref[...], w_ref[...],
                     preferred_element_type=jnp.float32,
                     precision=lax.Precision.HIGHEST)
        o_ref[...] = hm * dis

    return pl.pallas_call(
        body,
        grid=(n // _BN,),
        in_specs=[
            pl.BlockSpec((_BN, 1), lambda i: (i, 0)),
            pl.BlockSpec((_BN, 1), lambda i: (i, 0)),
            pl.BlockSpec((_BN, d), lambda i: (i, 0)),
            pl.BlockSpec((d, h), lambda i: (0, 0)),
        ],
        out_specs=pl.BlockSpec((_BN, h), lambda i: (i, 0)),
        out_shape=jax.ShapeDtypeStruct((n, h), jnp.float32),
    )(degp[0], degp[1], x, w)


def _mid_layer(aggp, hp, degp, b, w):
    n, d = hp.shape
    h = w.shape[1]

    def body(agg_ref, hp_ref, d0_ref, d1_ref, b_ref, w_ref, o_ref):
        dis = _dis_from(d0_ref, d1_ref)
        a = agg_ref[...]
        tot = a[0] + a[1] + hp_ref[...]
        x = jnp.maximum(tot * dis + b_ref[...], 0.0)
        hm = jnp.dot(x, w_ref[...],
                     preferred_element_type=jnp.float32,
                     precision=lax.Precision.HIGHEST)
        o_ref[...] = hm * dis

    return pl.pallas_call(
        body,
        grid=(n // _BN,),
        in_specs=[
            pl.BlockSpec((NC, _BN, d), lambda i: (0, i, 0)),
            pl.BlockSpec((_BN, d), lambda i: (i, 0)),
            pl.BlockSpec((_BN, 1), lambda i: (i, 0)),
            pl.BlockSpec((_BN, 1), lambda i: (i, 0)),
            pl.BlockSpec((1, d), lambda i: (0, 0)),
            pl.BlockSpec((d, h), lambda i: (0, 0)),
        ],
        out_specs=pl.BlockSpec((_BN, h), lambda i: (i, 0)),
        out_shape=jax.ShapeDtypeStruct((n, h), jnp.float32),
    )(aggp, hp, degp[0], degp[1], b, w)


def _final_layer(aggp, hp, degp, b):
    n, d = hp.shape

    def body(agg_ref, hp_ref, d0_ref, d1_ref, b_ref, o_ref):
        dis = _dis_from(d0_ref, d1_ref)
        a = agg_ref[...]
        tot = a[0] + a[1] + hp_ref[...]
        x = jnp.maximum(tot * dis + b_ref[...], 0.0)
        m = jnp.max(x, axis=0, keepdims=True)

        @pl.when(pl.program_id(0) == 0)
        def _init():
            o_ref[...] = jnp.zeros_like(o_ref)

        o_ref[...] = jnp.maximum(o_ref[...], m)

    return pl.pallas_call(
        body,
        grid=(n // _BN,),
        in_specs=[
            pl.BlockSpec((NC, _BN, d), lambda i: (0, i, 0)),
            pl.BlockSpec((_BN, d), lambda i: (i, 0)),
            pl.BlockSpec((_BN, 1), lambda i: (i, 0)),
            pl.BlockSpec((_BN, 1), lambda i: (i, 0)),
            pl.BlockSpec((1, d), lambda i: (0, 0)),
        ],
        out_specs=pl.BlockSpec((1, d), lambda i: (0, 0)),
        out_shape=jax.ShapeDtypeStruct((1, d), jnp.float32),
    )(aggp, hp, degp[0], degp[1], b)


def kernel(features, edge_index, edge_attr, W1, b1, W2, b2):
    n, d = features.shape
    e = edge_index.shape[1]
    assert e % (NW * _K) == 0 and n % NS == 0 and n % _BN == 0
    src = edge_index[0]
    dst = edge_index[1]
    ones_k = jnp.ones((_K,), jnp.float32)
    zeros_n = jnp.zeros((n,), jnp.float32)
    zeros_nd = jnp.zeros((n, d), jnp.float32)
    b1r = b1.reshape(1, -1)
    b2r = b2.reshape(1, -1)

    dp = _deg_partials(dst, ones_k, zeros_n, n)
    degp = (dp[0][:, None], dp[1][:, None])
    h1p = _scaled_matmul(degp, features, W1)
    agg1 = _agg_partials(h1p, src, dst, zeros_nd)
    h2p = _mid_layer(agg1, h1p, degp, b1r, W2)
    agg2 = _agg_partials(h2p, src, dst, zeros_nd)
    return _final_layer(agg2, h2p, degp, b2r)


# speedup vs baseline: 31.1969x; 1.0608x over previous
import functools

import jax
import jax.numpy as jnp
from jax import lax
from jax.experimental import pallas as pl
from jax.experimental.pallas import tpu as pltpu
from jax.experimental.pallas import tpu_sc as plsc

NC = 2
NS = 16
NW = NC * NS

_K = 112


def _deg_partials(dst, ones_k, zeros_n, n):
    e = dst.shape[0] - 2 * _K
    nfull = -(-e // _K)
    cbase = nfull // NW
    cextra = nfull % NW
    cmax = cbase + (1 if cextra else 0)
    mesh = plsc.VectorSubcoreMesh(core_axis_name="c", subcore_axis_name="s")

    @functools.partial(
        pl.kernel,
        out_type=[jax.ShapeDtypeStruct((n + 8,), jnp.float32)] * NC,
        mesh=mesh,
        scratch_types=[
            pltpu.VMEM((cmax * _K,), jnp.int32),
            pltpu.VMEM((_K,), jnp.int32),
            pltpu.VMEM((_K,), jnp.int32),
            pltpu.VMEM((_K,), jnp.float32),
            pltpu.VMEM_SHARED((n + 8,), jnp.float32),
            pltpu.SemaphoreType.DMA,
            pltpu.SemaphoreType.DMA,
        ],
    )
    def body(dst_hbm, ones_hbm, zeros_hbm, out0_hbm, out1_hbm,
             didx_all, didx0, didx1, ones_v, accum, ssem0, ssem1):
        c = lax.axis_index("c")
        s = lax.axis_index("s")
        wid = s * NC + c
        pltpu.sync_copy(ones_hbm, ones_v)

        @pl.when(s == 0)
        def _init():
            pltpu.sync_copy(zeros_hbm, accum)

        plsc.subcore_barrier()
        count = cbase + jnp.where(wid < cextra, 1, 0)
        start = wid * cbase + jnp.minimum(wid, cextra)
        base = start * _K
        pltpu.sync_copy(dst_hbm.at[pl.ds(base, cmax * _K)], didx_all)

        def stage_didx(i, didx_b):
            for j in range(_K // 16):
                didx_b[pl.ds(j * 16, 16)] = didx_all[pl.ds(i * _K + j * 16, 16)]

        def do_chunk(i, didx_b, ssem_b):
            @pl.when(i >= 2)
            def _free():
                pltpu.make_async_copy(ones_v, accum.at[didx_b], ssem_b).wait()

            stage_didx(i, didx_b)
            pltpu.async_copy(ones_v, accum.at[didx_b], ssem_b, add=True)

        @pl.loop(0, cmax)
        def _chunk(i):
            @pl.when((i < count) & (i % 2 == 0))
            def _even():
                do_chunk(i, didx0, ssem0)

            @pl.when((i < count) & (i % 2 == 1))
            def _odd():
                do_chunk(i, didx1, ssem1)

        pltpu.make_async_copy(ones_v, accum.at[didx0], ssem0).wait()
        pltpu.make_async_copy(ones_v, accum.at[didx1], ssem1).wait()
        plsc.subcore_barrier()

        @pl.when((s == 0) & (c == 0))
        def _flush0():
            pltpu.sync_copy(accum, out0_hbm)

        @pl.when((s == 0) & (c == 1))
        def _flush1():
            pltpu.sync_copy(accum, out1_hbm)

    return body(dst, ones_k, zeros_n)


def _agg_partials(hp, src, dst, zeros_nd):
    n, d = hp.shape
    e = src.shape[0] - 2 * _K
    nfull = -(-e // _K)
    cbase = nfull // NW
    cextra = nfull % NW
    cmax = cbase + (1 if cextra else 0)
    mesh = plsc.VectorSubcoreMesh(core_axis_name="c", subcore_axis_name="s")

    @functools.partial(
        pl.kernel,
        out_type=jax.ShapeDtypeStruct((NC, n + 8, d), jnp.float32),
        mesh=mesh,
        scratch_types=[
            pltpu.VMEM((cmax * _K,), jnp.int32),
            pltpu.VMEM((cmax * _K,), jnp.int32),
            pltpu.VMEM((_K,), jnp.int32),
            pltpu.VMEM((_K,), jnp.int32),
            pltpu.VMEM((_K, d), jnp.float32),
            pltpu.VMEM((_K, d), jnp.float32),
            pltpu.VMEM_SHARED((n + 8, d), jnp.float32),
            pltpu.SemaphoreType.DMA,
            pltpu.SemaphoreType.DMA,
            pltpu.SemaphoreType.DMA,
            pltpu.SemaphoreType.DMA,
        ],
    )
    def body(hp_hbm, src_hbm, dst_hbm, zeros_hbm, out_hbm,
             sidx_all, didx_all, didx0, didx1, rows0, rows1, accum,
             gsem0, gsem1, ssem0, ssem1):
        c = lax.axis_index("c")
        s = lax.axis_index("s")
        wid = s * NC + c

        @pl.when(s == 0)
        def _init():
            pltpu.sync_copy(zeros_hbm, accum)

        count = cbase + jnp.where(wid < cextra, 1, 0)
        start = wid * cbase + jnp.minimum(wid, cextra)
        base = start * _K
        pltpu.sync_copy(src_hbm.at[pl.ds(base, cmax * _K)], sidx_all)
        pltpu.sync_copy(dst_hbm.at[pl.ds(base, cmax * _K)], didx_all)
        plsc.subcore_barrier()

        def stage_didx(i, didx_b):
            for j in range(_K // 16):
                didx_b[pl.ds(j * 16, 16)] = didx_all[pl.ds(i * _K + j * 16, 16)]

        def issue(i, didx_b, rows_b, gsem_b, ssem_b):
            @pl.when(i >= 2)
            def _free():
                pltpu.make_async_copy(rows_b, accum.at[didx_b], ssem_b).wait()

            stage_didx(i, didx_b)
            pltpu.async_copy(hp_hbm.at[sidx_all.at[pl.ds(i * _K, _K)]],
                             rows_b, gsem_b)

        def complete(i, didx_b, rows_b, gsem_b, ssem_b):
            pltpu.make_async_copy(hp_hbm.at[sidx_all.at[pl.ds(i * _K, _K)]],
                                  rows_b, gsem_b).wait()
            pltpu.async_copy(rows_b, accum.at[didx_b], ssem_b, add=True)

        @pl.loop(0, cmax + 1)
        def _chunk(i):
            @pl.when((i < count) & (i % 2 == 0))
            def _issue_even():
                issue(i, didx0, rows0, gsem0, ssem0)

            @pl.when((i < count) & (i % 2 == 1))
            def _issue_odd():
                issue(i, didx1, rows1, gsem1, ssem1)

            @pl.when((i >= 1) & (i <= count) & (i % 2 == 1))
            def _complete_even():
                complete(i - 1, didx0, rows0, gsem0, ssem0)

            @pl.when((i >= 1) & (i <= count) & (i % 2 == 0))
            def _complete_odd():
                complete(i - 1, didx1, rows1, gsem1, ssem1)

        pltpu.make_async_copy(rows0, accum.at[didx0], ssem0).wait()
        pltpu.make_async_copy(rows1, accum.at[didx1], ssem1).wait()
        plsc.subcore_barrier()

        @pl.when(s == 0)
        def _flush():
            pltpu.sync_copy(accum, out_hbm.at[c])

    return body(hp, src, dst, zeros_nd)


def _dis_from(d0_ref, d1_ref):
    deg = 1.0 + d0_ref[...] + d1_ref[...]
    return lax.rsqrt(deg)


_BN = 1000


def _scaled_matmul(degp, x, w):
    n, d = x.shape
    h = w.shape[1]

    def body(d0_ref, d1_ref, x_ref, w_ref, o_ref):
        dis = _dis_from(d0_ref, d1_ref)
        hm = jnp.dot(x_ref[...], w_ref[...],
                     preferred_element_type=jnp.float32,
                     precision=lax.Precision.HIGHEST)
        o_ref[...] = hm * dis

    return pl.pallas_call(
        body,
        grid=(n // _BN,),
        in_specs=[
            pl.BlockSpec((_BN, 1), lambda i: (i, 0)),
            pl.BlockSpec((_BN, 1), lambda i: (i, 0)),
            pl.BlockSpec((_BN, d), lambda i: (i, 0)),
            pl.BlockSpec((d, h), lambda i: (0, 0)),
        ],
        out_specs=pl.BlockSpec((_BN, h), lambda i: (i, 0)),
        out_shape=jax.ShapeDtypeStruct((n, h), jnp.float32),
    )(degp[0], degp[1], x, w)


def _mid_layer(aggp, hp, degp, b, w):
    n, d = hp.shape
    h = w.shape[1]

    def body(agg_ref, hp_ref, d0_ref, d1_ref, b_ref, w_ref, o_ref):
        dis = _dis_from(d0_ref, d1_ref)
        a = agg_ref[...]
        tot = a[0] + a[1] + hp_ref[...]
        x = jnp.maximum(tot * dis + b_ref[...], 0.0)
        hm = jnp.dot(x, w_ref[...],
                     preferred_element_type=jnp.float32,
                     precision=lax.Precision.HIGHEST)
        o_ref[...] = hm * dis

    return pl.pallas_call(
        body,
        grid=(n // _BN,),
        in_specs=[
            pl.BlockSpec((NC, _BN, d), lambda i: (0, i, 0)),
            pl.BlockSpec((_BN, d), lambda i: (i, 0)),
            pl.BlockSpec((_BN, 1), lambda i: (i, 0)),
            pl.BlockSpec((_BN, 1), lambda i: (i, 0)),
            pl.BlockSpec((1, d), lambda i: (0, 0)),
            pl.BlockSpec((d, h), lambda i: (0, 0)),
        ],
        out_specs=pl.BlockSpec((_BN, h), lambda i: (i, 0)),
        out_shape=jax.ShapeDtypeStruct((n, h), jnp.float32),
    )(aggp, hp, degp[0], degp[1], b, w)


def _final_layer(aggp, hp, degp, b):
    n, d = hp.shape

    def body(agg_ref, hp_ref, d0_ref, d1_ref, b_ref, o_ref):
        dis = _dis_from(d0_ref, d1_ref)
        a = agg_ref[...]
        tot = a[0] + a[1] + hp_ref[...]
        x = jnp.maximum(tot * dis + b_ref[...], 0.0)
        m = jnp.max(x, axis=0, keepdims=True)

        @pl.when(pl.program_id(0) == 0)
        def _init():
            o_ref[...] = jnp.zeros_like(o_ref)

        o_ref[...] = jnp.maximum(o_ref[...], m)

    return pl.pallas_call(
        body,
        grid=(n // _BN,),
        in_specs=[
            pl.BlockSpec((NC, _BN, d), lambda i: (0, i, 0)),
            pl.BlockSpec((_BN, d), lambda i: (i, 0)),
            pl.BlockSpec((_BN, 1), lambda i: (i, 0)),
            pl.BlockSpec((_BN, 1), lambda i: (i, 0)),
            pl.BlockSpec((1, d), lambda i: (0, 0)),
        ],
        out_specs=pl.BlockSpec((1, d), lambda i: (0, 0)),
        out_shape=jax.ShapeDtypeStruct((1, d), jnp.float32),
    )(aggp, hp, degp[0], degp[1], b)


def kernel(features, edge_index, edge_attr, W1, b1, W2, b2):
    n, d = features.shape
    e = edge_index.shape[1]
    assert n % _BN == 0
    src = jnp.pad(edge_index[0], (0, 2 * _K))
    dst = jnp.pad(edge_index[1], (0, 2 * _K),
                  constant_values=n)
    ones_k = jnp.ones((_K,), jnp.float32)
    zeros_n = jnp.zeros((n + 8,), jnp.float32)
    zeros_nd = jnp.zeros((n + 8, d), jnp.float32)
    b1r = b1.reshape(1, -1)
    b2r = b2.reshape(1, -1)

    dp = _deg_partials(dst, ones_k, zeros_n, n)
    degp = (dp[0][:, None], dp[1][:, None])
    h1p = _scaled_matmul(degp, features, W1)
    agg1 = _agg_partials(h1p, src, dst, zeros_nd)
    h2p = _mid_layer(agg1, h1p, degp, b1r, W2)
    agg2 = _agg_partials(h2p, src, dst, zeros_nd)
    return _final_layer(agg2, h2p, degp, b2r)


# speedup vs baseline: 32.1424x; 1.0303x over previous
import functools

import jax
import jax.numpy as jnp
from jax import lax
from jax.experimental import pallas as pl
from jax.experimental.pallas import tpu as pltpu
from jax.experimental.pallas import tpu_sc as plsc

NC = 2
NS = 16
NW = NC * NS

_K = 112


def _deg_partials(dst, ones_k, n):
    e = dst.shape[0] - 2 * _K
    nfull = -(-e // _K)
    cbase = nfull // NW
    cextra = nfull % NW
    cmax = cbase + (1 if cextra else 0)
    mesh = plsc.VectorSubcoreMesh(core_axis_name="c", subcore_axis_name="s")

    rpt = -(-(n + 8) // NS // 8) * 8
    npad = NS * rpt
    rfull = rpt // _K
    rrem = rpt - rfull * _K

    @functools.partial(
        pl.kernel,
        out_type=[jax.ShapeDtypeStruct((npad,), jnp.float32)] * NC,
        mesh=mesh,
        scratch_types=[
            pltpu.VMEM((cmax * _K,), jnp.int32),
            pltpu.VMEM((_K,), jnp.int32),
            pltpu.VMEM((_K,), jnp.int32),
            pltpu.VMEM((_K,), jnp.float32),
            pltpu.VMEM((_K,), jnp.float32),
            pltpu.VMEM_SHARED((npad,), jnp.float32),
            pltpu.SemaphoreType.DMA,
            pltpu.SemaphoreType.DMA,
        ],
    )
    def body(dst_hbm, ones_hbm, out0_hbm, out1_hbm,
             didx_all, didx0, didx1, ones_v, zbuf, accum, ssem0, ssem1):
        c = lax.axis_index("c")
        s = lax.axis_index("s")
        wid = s * NC + c
        pltpu.sync_copy(ones_hbm, ones_v)

        for j in range(_K // 16):
            zbuf[pl.ds(j * 16, 16)] = jnp.zeros((16,), jnp.float32)
        for r in range(rfull):
            pltpu.sync_copy(zbuf, accum.at[pl.ds(s * rpt + r * _K, _K)])
        if rrem:
            pltpu.sync_copy(zbuf.at[pl.ds(0, rrem)],
                            accum.at[pl.ds(s * rpt + rfull * _K, rrem)])

        plsc.subcore_barrier()
        count = cbase + jnp.where(wid < cextra, 1, 0)
        start = wid * cbase + jnp.minimum(wid, cextra)
        base = start * _K
        pltpu.sync_copy(dst_hbm.at[pl.ds(base, cmax * _K)], didx_all)

        def stage_didx(i, didx_b):
            for j in range(_K // 16):
                didx_b[pl.ds(j * 16, 16)] = didx_all[pl.ds(i * _K + j * 16, 16)]

        def do_chunk(i, didx_b, ssem_b):
            @pl.when(i >= 2)
            def _free():
                pltpu.make_async_copy(ones_v, accum.at[didx_b], ssem_b).wait()

            stage_didx(i, didx_b)
            pltpu.async_copy(ones_v, accum.at[didx_b], ssem_b, add=True)

        @pl.loop(0, cmax)
        def _chunk(i):
            @pl.when((i < count) & (i % 2 == 0))
            def _even():
                do_chunk(i, didx0, ssem0)

            @pl.when((i < count) & (i % 2 == 1))
            def _odd():
                do_chunk(i, didx1, ssem1)

        pltpu.make_async_copy(ones_v, accum.at[didx0], ssem0).wait()
        pltpu.make_async_copy(ones_v, accum.at[didx1], ssem1).wait()
        plsc.subcore_barrier()

        @pl.when((s == 0) & (c == 0))
        def _flush0():
            pltpu.sync_copy(accum, out0_hbm)

        @pl.when((s == 0) & (c == 1))
        def _flush1():
            pltpu.sync_copy(accum, out1_hbm)

    return body(dst, ones_k)


def _agg_partials(hp, src, dst):
    n, d = hp.shape
    e = src.shape[0] - 2 * _K
    nfull = -(-e // _K)
    cbase = nfull // NW
    cextra = nfull % NW
    cmax = cbase + (1 if cextra else 0)
    mesh = plsc.VectorSubcoreMesh(core_axis_name="c", subcore_axis_name="s")

    rpt = -(-(n + 8) // NS // 8) * 8
    npad = NS * rpt
    rfull = rpt // _K
    rrem = rpt - rfull * _K

    @functools.partial(
        pl.kernel,
        out_type=jax.ShapeDtypeStruct((NC, npad, d), jnp.float32),
        mesh=mesh,
        scratch_types=[
            pltpu.VMEM((cmax * _K,), jnp.int32),
            pltpu.VMEM((cmax * _K,), jnp.int32),
            pltpu.VMEM((_K,), jnp.int32),
            pltpu.VMEM((_K,), jnp.int32),
            pltpu.VMEM((_K, d), jnp.float32),
            pltpu.VMEM((_K, d), jnp.float32),
            pltpu.VMEM_SHARED((npad, d), jnp.float32),
            pltpu.SemaphoreType.DMA,
            pltpu.SemaphoreType.DMA,
            pltpu.SemaphoreType.DMA,
            pltpu.SemaphoreType.DMA,
        ],
    )
    def body(hp_hbm, src_hbm, dst_hbm, out_hbm,
             sidx_all, didx_all, didx0, didx1, rows0, rows1, accum,
             gsem0, gsem1, ssem0, ssem1):
        c = lax.axis_index("c")
        s = lax.axis_index("s")
        wid = s * NC + c

        @pl.loop(0, _K)
        def _zfill(i):
            for j in range(d // 16):
                rows0[i, pl.ds(j * 16, 16)] = jnp.zeros((16,), jnp.float32)
        for r in range(rfull):
            pltpu.sync_copy(rows0, accum.at[pl.ds(s * rpt + r * _K, _K)])
        if rrem:
            pltpu.sync_copy(rows0.at[pl.ds(0, rrem)],
                            accum.at[pl.ds(s * rpt + rfull * _K, rrem)])

        count = cbase + jnp.where(wid < cextra, 1, 0)
        start = wid * cbase + jnp.minimum(wid, cextra)
        base = start * _K
        pltpu.sync_copy(src_hbm.at[pl.ds(base, cmax * _K)], sidx_all)
        pltpu.sync_copy(dst_hbm.at[pl.ds(base, cmax * _K)], didx_all)
        plsc.subcore_barrier()

        def stage_didx(i, didx_b):
            for j in range(_K // 16):
                didx_b[pl.ds(j * 16, 16)] = didx_all[pl.ds(i * _K + j * 16, 16)]

        def issue(i, didx_b, rows_b, gsem_b, ssem_b):
            @pl.when(i >= 2)
            def _free():
                pltpu.make_async_copy(rows_b, accum.at[didx_b], ssem_b).wait()

            stage_didx(i, didx_b)
            pltpu.async_copy(hp_hbm.at[sidx_all.at[pl.ds(i * _K, _K)]],
                             rows_b, gsem_b)

        def complete(i, didx_b, rows_b, gsem_b, ssem_b):
            pltpu.make_async_copy(hp_hbm.at[sidx_all.at[pl.ds(i * _K, _K)]],
                                  rows_b, gsem_b).wait()
            pltpu.async_copy(rows_b, accum.at[didx_b], ssem_b, add=True)

        @pl.loop(0, cmax + 1)
        def _chunk(i):
            @pl.when((i < count) & (i % 2 == 0))
            def _issue_even():
                issue(i, didx0, rows0, gsem0, ssem0)

            @pl.when((i < count) & (i % 2 == 1))
            def _issue_odd():
                issue(i, didx1, rows1, gsem1, ssem1)

            @pl.when((i >= 1) & (i <= count) & (i % 2 == 1))
            def _complete_even():
                complete(i - 1, didx0, rows0, gsem0, ssem0)

            @pl.when((i >= 1) & (i <= count) & (i % 2 == 0))
            def _complete_odd():
                complete(i - 1, didx1, rows1, gsem1, ssem1)

        pltpu.make_async_copy(rows0, accum.at[didx0], ssem0).wait()
        pltpu.make_async_copy(rows1, accum.at[didx1], ssem1).wait()
        plsc.subcore_barrier()

        @pl.when(s == 0)
        def _flush():
            pltpu.sync_copy(accum, out_hbm.at[c])

    return body(hp, src, dst)


def _dis_from(d0_ref, d1_ref):
    deg = 1.0 + d0_ref[...] + d1_ref[...]
    return lax.rsqrt(deg)


_BN = 1000


def _scaled_matmul(degp, x, w):
    n, d = x.shape
    h = w.shape[1]

    def body(d0_ref, d1_ref, x_ref, w_ref, o_ref, db_ref):
        dis = _dis_from(d0_ref, d1_ref)
        hm = jnp.dot(x_ref[...], w_ref[...],
                     preferred_element_type=jnp.float32,
                     precision=lax.Precision.HIGHEST)
        o_ref[...] = hm * dis
        db_ref[...] = jnp.broadcast_to(dis, (dis.shape[0], h))

    return pl.pallas_call(
        body,
        grid=(n // _BN,),
        in_specs=[
            pl.BlockSpec((_BN, 1), lambda i: (i, 0)),
            pl.BlockSpec((_BN, 1), lambda i: (i, 0)),
            pl.BlockSpec((_BN, d), lambda i: (i, 0)),
            pl.BlockSpec((d, h), lambda i: (0, 0)),
        ],
        out_specs=[pl.BlockSpec((_BN, h), lambda i: (i, 0)),
                   pl.BlockSpec((_BN, h), lambda i: (i, 0))],
        out_shape=[jax.ShapeDtypeStruct((n, h), jnp.float32),
                   jax.ShapeDtypeStruct((n, h), jnp.float32)],
    )(degp[0], degp[1], x, w)


def _mid_layer(aggp, hp, disb, b, w):
    n, d = hp.shape
    h = w.shape[1]

    def body(agg_ref, hp_ref, db_ref, b_ref, w_ref, o_ref):
        dis = db_ref[...]
        a = agg_ref[...]
        tot = a[0] + a[1] + hp_ref[...]
        x = jnp.maximum(tot * dis + b_ref[...], 0.0)
        hm = jnp.dot(x, w_ref[...],
                     preferred_element_type=jnp.float32,
                     precision=lax.Precision.HIGHEST)
        o_ref[...] = hm * dis

    return pl.pallas_call(
        body,
        grid=(n // _BN,),
        in_specs=[
            pl.BlockSpec((NC, _BN, d), lambda i: (0, i, 0)),
            pl.BlockSpec((_BN, d), lambda i: (i, 0)),
            pl.BlockSpec((_BN, d), lambda i: (i, 0)),
            pl.BlockSpec((1, d), lambda i: (0, 0)),
            pl.BlockSpec((d, h), lambda i: (0, 0)),
        ],
        out_specs=pl.BlockSpec((_BN, h), lambda i: (i, 0)),
        out_shape=jax.ShapeDtypeStruct((n, h), jnp.float32),
    )(aggp, hp, disb, b, w)


def _final_layer(aggp, hp, disb, b):
    n, d = hp.shape

    def body(agg_ref, hp_ref, db_ref, b_ref, o_ref):
        dis = db_ref[...]
        a = agg_ref[...]
        tot = a[0] + a[1] + hp_ref[...]
        x = jnp.maximum(tot * dis + b_ref[...], 0.0)
        m = jnp.max(x, axis=0, keepdims=True)

        @pl.when(pl.program_id(0) == 0)
        def _init():
            o_ref[...] = jnp.zeros_like(o_ref)

        o_ref[...] = jnp.maximum(o_ref[...], m)

    return pl.pallas_call(
        body,
        grid=(n // _BN,),
        in_specs=[
            pl.BlockSpec((NC, _BN, d), lambda i: (0, i, 0)),
            pl.BlockSpec((_BN, d), lambda i: (i, 0)),
            pl.BlockSpec((_BN, d), lambda i: (i, 0)),
            pl.BlockSpec((1, d), lambda i: (0, 0)),
        ],
        out_specs=pl.BlockSpec((1, d), lambda i: (0, 0)),
        out_shape=jax.ShapeDtypeStruct((1, d), jnp.float32),
    )(aggp, hp, disb, b)


def kernel(features, edge_index, edge_attr, W1, b1, W2, b2):
    n, d = features.shape
    e = edge_index.shape[1]
    assert n % _BN == 0
    src = jnp.pad(edge_index[0], (0, 2 * _K))
    dst = jnp.pad(edge_index[1], (0, 2 * _K),
                  constant_values=n)
    ones_k = jnp.ones((_K,), jnp.float32)
    b1r = b1.reshape(1, -1)
    b2r = b2.reshape(1, -1)

    dp = _deg_partials(dst, ones_k, n)
    degp = (dp[0][:, None], dp[1][:, None])
    h1p, disb = _scaled_matmul(degp, features, W1)
    agg1 = _agg_partials(h1p, src, dst)
    h2p = _mid_layer(agg1, h1p, disb, b1r, W2)
    agg2 = _agg_partials(h2p, src, dst)
    return _final_layer(agg2, h2p, disb, b2r)


# speedup vs baseline: 33.8111x; 1.0519x over previous
import functools

import jax
import jax.numpy as jnp
from jax import lax
from jax.experimental import pallas as pl
from jax.experimental.pallas import tpu as pltpu
from jax.experimental.pallas import tpu_sc as plsc

NC = 2
NS = 16
NW = NC * NS

_K = 112


def _deg_partials(dst, ones_k, n):
    e = dst.shape[0] - 2 * _K
    nfull = -(-e // _K)
    cbase = nfull // NW
    cextra = nfull % NW
    cmax = cbase + (1 if cextra else 0)
    mesh = plsc.VectorSubcoreMesh(core_axis_name="c", subcore_axis_name="s")

    rpt = -(-(n + 8) // NS // 8) * 8
    npad = NS * rpt
    rfull = rpt // _K
    rrem = rpt - rfull * _K

    @functools.partial(
        pl.kernel,
        out_type=[jax.ShapeDtypeStruct((npad,), jnp.float32)] * NC,
        mesh=mesh,
        scratch_types=[
            pltpu.VMEM((cmax * _K,), jnp.int32),
            pltpu.VMEM((_K,), jnp.int32),
            pltpu.VMEM((_K,), jnp.int32),
            pltpu.VMEM((_K,), jnp.float32),
            pltpu.VMEM((_K,), jnp.float32),
            pltpu.VMEM_SHARED((npad,), jnp.float32),
            pltpu.SemaphoreType.DMA,
            pltpu.SemaphoreType.DMA,
        ],
    )
    def body(dst_hbm, ones_hbm, out0_hbm, out1_hbm,
             didx_all, didx0, didx1, ones_v, zbuf, accum, ssem0, ssem1):
        c = lax.axis_index("c")
        s = lax.axis_index("s")
        wid = s * NC + c
        pltpu.sync_copy(ones_hbm, ones_v)

        for j in range(_K // 16):
            zbuf[pl.ds(j * 16, 16)] = jnp.zeros((16,), jnp.float32)
        for r in range(rfull):
            pltpu.sync_copy(zbuf, accum.at[pl.ds(s * rpt + r * _K, _K)])
        if rrem:
            pltpu.sync_copy(zbuf.at[pl.ds(0, rrem)],
                            accum.at[pl.ds(s * rpt + rfull * _K, rrem)])

        plsc.subcore_barrier()
        count = cbase + jnp.where(wid < cextra, 1, 0)
        start = wid * cbase + jnp.minimum(wid, cextra)
        base = start * _K
        pltpu.sync_copy(dst_hbm.at[pl.ds(base, cmax * _K)], didx_all)

        def stage_didx(i, didx_b):
            for j in range(_K // 16):
                didx_b[pl.ds(j * 16, 16)] = didx_all[pl.ds(i * _K + j * 16, 16)]

        def do_chunk(i, didx_b, ssem_b):
            @pl.when(i >= 2)
            def _free():
                pltpu.make_async_copy(ones_v, accum.at[didx_b], ssem_b).wait()

            stage_didx(i, didx_b)
            pltpu.async_copy(ones_v, accum.at[didx_b], ssem_b, add=True)

        @pl.loop(0, cmax)
        def _chunk(i):
            @pl.when((i < count) & (i % 2 == 0))
            def _even():
                do_chunk(i, didx0, ssem0)

            @pl.when((i < count) & (i % 2 == 1))
            def _odd():
                do_chunk(i, didx1, ssem1)

        pltpu.make_async_copy(ones_v, accum.at[didx0], ssem0).wait()
        pltpu.make_async_copy(ones_v, accum.at[didx1], ssem1).wait()
        plsc.subcore_barrier()

        @pl.when((s == 0) & (c == 0))
        def _flush0():
            pltpu.sync_copy(accum, out0_hbm)

        @pl.when((s == 0) & (c == 1))
        def _flush1():
            pltpu.sync_copy(accum, out1_hbm)

    return body(dst, ones_k)


def _agg_partials(hp, src, dst):
    n, d = hp.shape
    e = src.shape[0] - 2 * _K
    nfull = -(-e // _K)
    cbase = nfull // NW
    cextra = nfull % NW
    cmax = cbase + (1 if cextra else 0)
    mesh = plsc.VectorSubcoreMesh(core_axis_name="c", subcore_axis_name="s")

    rpt = -(-(n + 8) // NS // 8) * 8
    npad = NS * rpt
    rfull = rpt // _K
    rrem = rpt - rfull * _K

    @functools.partial(
        pl.kernel,
        out_type=jax.ShapeDtypeStruct((NC, npad, d), jnp.float32),
        mesh=mesh,
        scratch_types=[
            pltpu.VMEM((cmax * _K,), jnp.int32),
            pltpu.VMEM((cmax * _K,), jnp.int32),
            pltpu.VMEM((_K,), jnp.int32),
            pltpu.VMEM((_K,), jnp.int32),
            pltpu.VMEM((_K, d), jnp.float32),
            pltpu.VMEM((_K, d), jnp.float32),
            pltpu.VMEM_SHARED((npad, d), jnp.float32),
            pltpu.SemaphoreType.DMA,
            pltpu.SemaphoreType.DMA,
            pltpu.SemaphoreType.DMA,
            pltpu.SemaphoreType.DMA,
        ],
    )
    def body(hp_hbm, src_hbm, dst_hbm, out_hbm,
             sidx_all, didx_all, didx0, didx1, rows0, rows1, accum,
             gsem0, gsem1, ssem0, ssem1):
        c = lax.axis_index("c")
        s = lax.axis_index("s")
        wid = s * NC + c

        @pl.loop(0, _K)
        def _zfill(i):
            for j in range(d // 16):
                rows0[i, pl.ds(j * 16, 16)] = jnp.zeros((16,), jnp.float32)
        for r in range(rfull):
            pltpu.sync_copy(rows0, accum.at[pl.ds(s * rpt + r * _K, _K)])
        if rrem:
            pltpu.sync_copy(rows0.at[pl.ds(0, rrem)],
                            accum.at[pl.ds(s * rpt + rfull * _K, rrem)])

        count = cbase + jnp.where(wid < cextra, 1, 0)
        start = wid * cbase + jnp.minimum(wid, cextra)
        base = start * _K
        pltpu.sync_copy(src_hbm.at[pl.ds(base, cmax * _K)], sidx_all)
        pltpu.sync_copy(dst_hbm.at[pl.ds(base, cmax * _K)], didx_all)
        plsc.subcore_barrier()

        def stage_didx(i, didx_b):
            for j in range(_K // 16):
                didx_b[pl.ds(j * 16, 16)] = didx_all[pl.ds(i * _K + j * 16, 16)]

        def issue(i, didx_b, rows_b, gsem_b, ssem_b):
            @pl.when(i >= 2)
            def _free():
                pltpu.make_async_copy(rows_b, accum.at[didx_b], ssem_b).wait()

            stage_didx(i, didx_b)
            pltpu.async_copy(hp_hbm.at[sidx_all.at[pl.ds(i * _K, _K)]],
                             rows_b, gsem_b)

        def complete(i, didx_b, rows_b, gsem_b, ssem_b):
            pltpu.make_async_copy(hp_hbm.at[sidx_all.at[pl.ds(i * _K, _K)]],
                                  rows_b, gsem_b).wait()
            pltpu.async_copy(rows_b, accum.at[didx_b], ssem_b, add=True)

        @pl.loop(0, cmax + 1)
        def _chunk(i):
            @pl.when((i < count) & (i % 2 == 0))
            def _issue_even():
                issue(i, didx0, rows0, gsem0, ssem0)

            @pl.when((i < count) & (i % 2 == 1))
            def _issue_odd():
                issue(i, didx1, rows1, gsem1, ssem1)

            @pl.when((i >= 1) & (i <= count) & (i % 2 == 1))
            def _complete_even():
                complete(i - 1, didx0, rows0, gsem0, ssem0)

            @pl.when((i >= 1) & (i <= count) & (i % 2 == 0))
            def _complete_odd():
                complete(i - 1, didx1, rows1, gsem1, ssem1)

        pltpu.make_async_copy(rows0, accum.at[didx0], ssem0).wait()
        pltpu.make_async_copy(rows1, accum.at[didx1], ssem1).wait()
        plsc.subcore_barrier()

        @pl.when(s == 0)
        def _flush():
            pltpu.sync_copy(accum, out_hbm.at[c])

    return body(hp, src, dst)


def _dis_from(d0_ref, d1_ref):
    deg = 1.0 + d0_ref[...] + d1_ref[...]
    return lax.rsqrt(deg)


_BN = 1000


def _scaled_matmul(dp, x, w):
    n, d = x.shape
    h = w.shape[1]
    npad = dp[0].shape[0]

    def body(d0_ref, d1_ref, x_ref, w_ref, o_ref, db_ref):
        deg = 1.0 + d0_ref[...] + d1_ref[...]
        dis2 = lax.rsqrt(jnp.reshape(deg, (npad, 1)))
        dis = dis2[:n, :]
        hm = jnp.dot(x_ref[...], w_ref[...],
                     preferred_element_type=jnp.float32,
                     precision=lax.Precision.HIGHEST)
        o_ref[...] = hm * dis
        db_ref[...] = jnp.broadcast_to(dis, (n, h))

    return pl.pallas_call(
        body,
        in_specs=[
            pl.BlockSpec((npad,), lambda: (0,)),
            pl.BlockSpec((npad,), lambda: (0,)),
            pl.BlockSpec((n, d), lambda: (0, 0)),
            pl.BlockSpec((d, h), lambda: (0, 0)),
        ],
        out_specs=[pl.BlockSpec((n, h), lambda: (0, 0)),
                   pl.BlockSpec((n, h), lambda: (0, 0))],
        out_shape=[jax.ShapeDtypeStruct((n, h), jnp.float32),
                   jax.ShapeDtypeStruct((n, h), jnp.float32)],
    )(dp[0], dp[1], x, w)


def _mid_layer(aggp, hp, disb, b, w):
    n, d = hp.shape
    h = w.shape[1]

    def body(agg_ref, hp_ref, db_ref, b_ref, w_ref, o_ref):
        dis = db_ref[...]
        a = agg_ref[...]
        tot = a[0] + a[1] + hp_ref[...]
        x = jnp.maximum(tot * dis + b_ref[...], 0.0)
        hm = jnp.dot(x, w_ref[...],
                     preferred_element_type=jnp.float32,
                     precision=lax.Precision.HIGHEST)
        o_ref[...] = hm * dis

    return pl.pallas_call(
        body,
        grid=(n // _BN,),
        in_specs=[
            pl.BlockSpec((NC, _BN, d), lambda i: (0, i, 0)),
            pl.BlockSpec((_BN, d), lambda i: (i, 0)),
            pl.BlockSpec((_BN, d), lambda i: (i, 0)),
            pl.BlockSpec((1, d), lambda i: (0, 0)),
            pl.BlockSpec((d, h), lambda i: (0, 0)),
        ],
        out_specs=pl.BlockSpec((_BN, h), lambda i: (i, 0)),
        out_shape=jax.ShapeDtypeStruct((n, h), jnp.float32),
    )(aggp, hp, disb, b, w)


def _final_layer(aggp, hp, disb, b):
    n, d = hp.shape

    def body(agg_ref, hp_ref, db_ref, b_ref, o_ref):
        dis = db_ref[...]
        a = agg_ref[...]
        tot = a[0] + a[1] + hp_ref[...]
        x = jnp.maximum(tot * dis + b_ref[...], 0.0)
        m = jnp.max(x, axis=0, keepdims=True)

        @pl.when(pl.program_id(0) == 0)
        def _init():
            o_ref[...] = jnp.zeros_like(o_ref)

        o_ref[...] = jnp.maximum(o_ref[...], m)

    return pl.pallas_call(
        body,
        grid=(n // _BN,),
        in_specs=[
            pl.BlockSpec((NC, _BN, d), lambda i: (0, i, 0)),
            pl.BlockSpec((_BN, d), lambda i: (i, 0)),
            pl.BlockSpec((_BN, d), lambda i: (i, 0)),
            pl.BlockSpec((1, d), lambda i: (0, 0)),
        ],
        out_specs=pl.BlockSpec((1, d), lambda i: (0, 0)),
        out_shape=jax.ShapeDtypeStruct((1, d), jnp.float32),
    )(aggp, hp, disb, b)


def kernel(features, edge_index, edge_attr, W1, b1, W2, b2):
    n, d = features.shape
    e = edge_index.shape[1]
    assert n % _BN == 0
    src = jnp.pad(edge_index[0], (0, 2 * _K))
    dst = jnp.pad(edge_index[1], (0, 2 * _K),
                  constant_values=n)
    ones_k = jnp.ones((_K,), jnp.float32)
    b1r = b1.reshape(1, -1)
    b2r = b2.reshape(1, -1)

    dp = _deg_partials(dst, ones_k, n)
    h1p, disb = _scaled_matmul(dp, features, W1)
    agg1 = _agg_partials(h1p, src, dst)
    h2p = _mid_layer(agg1, h1p, disb, b1r, W2)
    agg2 = _agg_partials(h2p, src, dst)
    return _final_layer(agg2, h2p, disb, b2r)


# speedup vs baseline: 33.9833x; 1.0051x over previous
import functools

import jax
import jax.numpy as jnp
from jax import lax
from jax.experimental import pallas as pl
from jax.experimental.pallas import tpu as pltpu
from jax.experimental.pallas import tpu_sc as plsc

NC = 2
NS = 16
NW = NC * NS

_K = 112


def _deg_partials(dst, ones_k, n):
    e = dst.shape[0] - 2 * _K
    nfull = -(-e // _K)
    cbase = nfull // NW
    cextra = nfull % NW
    cmax = cbase + (1 if cextra else 0)
    mesh = plsc.VectorSubcoreMesh(core_axis_name="c", subcore_axis_name="s")

    rpt = -(-(n + 8) // NS // 8) * 8
    npad = NS * rpt
    rfull = rpt // _K
    rrem = rpt - rfull * _K

    @functools.partial(
        pl.kernel,
        out_type=[jax.ShapeDtypeStruct((npad,), jnp.float32)] * NC,
        mesh=mesh,
        scratch_types=[
            pltpu.VMEM((cmax * _K,), jnp.int32),
            pltpu.VMEM((_K,), jnp.int32),
            pltpu.VMEM((_K,), jnp.int32),
            pltpu.VMEM((_K,), jnp.float32),
            pltpu.VMEM((_K,), jnp.float32),
            pltpu.VMEM_SHARED((npad,), jnp.float32),
            pltpu.SemaphoreType.DMA,
            pltpu.SemaphoreType.DMA,
        ],
    )
    def body(dst_hbm, ones_hbm, out0_hbm, out1_hbm,
             didx_all, didx0, didx1, ones_v, zbuf, accum, ssem0, ssem1):
        c = lax.axis_index("c")
        s = lax.axis_index("s")
        wid = s * NC + c
        pltpu.sync_copy(ones_hbm, ones_v)

        for j in range(_K // 16):
            zbuf[pl.ds(j * 16, 16)] = jnp.zeros((16,), jnp.float32)
        for r in range(rfull):
            pltpu.sync_copy(zbuf, accum.at[pl.ds(s * rpt + r * _K, _K)])
        if rrem:
            pltpu.sync_copy(zbuf.at[pl.ds(0, rrem)],
                            accum.at[pl.ds(s * rpt + rfull * _K, rrem)])

        plsc.subcore_barrier()
        count = cbase + jnp.where(wid < cextra, 1, 0)
        start = wid * cbase + jnp.minimum(wid, cextra)
        base = start * _K
        pltpu.sync_copy(dst_hbm.at[pl.ds(base, cmax * _K)], didx_all)

        def stage_didx(i, didx_b):
            for j in range(_K // 16):
                didx_b[pl.ds(j * 16, 16)] = didx_all[pl.ds(i * _K + j * 16, 16)]

        def do_chunk(i, didx_b, ssem_b):
            @pl.when(i >= 2)
            def _free():
                pltpu.make_async_copy(ones_v, accum.at[didx_b], ssem_b).wait()

            stage_didx(i, didx_b)
            pltpu.async_copy(ones_v, accum.at[didx_b], ssem_b, add=True)

        @pl.loop(0, cmax)
        def _chunk(i):
            @pl.when((i < count) & (i % 2 == 0))
            def _even():
                do_chunk(i, didx0, ssem0)

            @pl.when((i < count) & (i % 2 == 1))
            def _odd():
                do_chunk(i, didx1, ssem1)

        pltpu.make_async_copy(ones_v, accum.at[didx0], ssem0).wait()
        pltpu.make_async_copy(ones_v, accum.at[didx1], ssem1).wait()
        plsc.subcore_barrier()

        @pl.when((s == 0) & (c == 0))
        def _flush0():
            pltpu.sync_copy(accum, out0_hbm)

        @pl.when((s == 0) & (c == 1))
        def _flush1():
            pltpu.sync_copy(accum, out1_hbm)

    return body(dst, ones_k)


def _agg_partials(hp, src, dst):
    n, d = hp.shape
    e = src.shape[0] - 2 * _K
    nfull = -(-e // _K)
    cbase = nfull // NW
    cextra = nfull % NW
    cmax = cbase + (1 if cextra else 0)
    mesh = plsc.VectorSubcoreMesh(core_axis_name="c", subcore_axis_name="s")

    rpt = -(-(n + 8) // NS // 8) * 8
    npad = NS * rpt
    rfull = rpt // _K
    rrem = rpt - rfull * _K

    @functools.partial(
        pl.kernel,
        out_type=jax.ShapeDtypeStruct((NC, npad, d), jnp.float32),
        mesh=mesh,
        scratch_types=[
            pltpu.VMEM((cmax * _K,), jnp.int32),
            pltpu.VMEM((cmax * _K,), jnp.int32),
            pltpu.VMEM((_K,), jnp.int32),
            pltpu.VMEM((_K,), jnp.int32),
            pltpu.VMEM((_K, d), jnp.float32),
            pltpu.VMEM((_K, d), jnp.float32),
            pltpu.VMEM_SHARED((npad, d), jnp.float32),
            pltpu.SemaphoreType.DMA,
            pltpu.SemaphoreType.DMA,
            pltpu.SemaphoreType.DMA,
            pltpu.SemaphoreType.DMA,
        ],
    )
    def body(hp_hbm, src_hbm, dst_hbm, out_hbm,
             sidx_all, didx_all, didx0, didx1, rows0, rows1, accum,
             gsem0, gsem1, ssem0, ssem1):
        c = lax.axis_index("c")
        s = lax.axis_index("s")
        wid = s * NC + c

        @pl.loop(0, _K)
        def _zfill(i):
            for j in range(d // 16):
                rows0[i, pl.ds(j * 16, 16)] = jnp.zeros((16,), jnp.float32)
        for r in range(rfull):
            pltpu.sync_copy(rows0, accum.at[pl.ds(s * rpt + r * _K, _K)])
        if rrem:
            pltpu.sync_copy(rows0.at[pl.ds(0, rrem)],
                            accum.at[pl.ds(s * rpt + rfull * _K, rrem)])

        count = cbase + jnp.where(wid < cextra, 1, 0)
        start = wid * cbase + jnp.minimum(wid, cextra)
        base = start * _K
        pltpu.sync_copy(src_hbm.at[pl.ds(base, cmax * _K)], sidx_all)
        pltpu.sync_copy(dst_hbm.at[pl.ds(base, cmax * _K)], didx_all)
        plsc.subcore_barrier()

        def stage_didx(i, didx_b):
            for j in range(_K // 16):
                didx_b[pl.ds(j * 16, 16)] = didx_all[pl.ds(i * _K + j * 16, 16)]

        def issue(i, didx_b, rows_b, gsem_b, ssem_b):
            @pl.when(i >= 2)
            def _free():
                pltpu.make_async_copy(rows_b, accum.at[didx_b], ssem_b).wait()

            stage_didx(i, didx_b)
            pltpu.async_copy(hp_hbm.at[sidx_all.at[pl.ds(i * _K, _K)]],
                             rows_b, gsem_b)

        def complete(i, didx_b, rows_b, gsem_b, ssem_b):
            pltpu.make_async_copy(hp_hbm.at[sidx_all.at[pl.ds(i * _K, _K)]],
                                  rows_b, gsem_b).wait()
            pltpu.async_copy(rows_b, accum.at[didx_b], ssem_b, add=True)

        @pl.loop(0, cmax + 1)
        def _chunk(i):
            @pl.when((i < count) & (i % 2 == 0))
            def _issue_even():
                issue(i, didx0, rows0, gsem0, ssem0)

            @pl.when((i < count) & (i % 2 == 1))
            def _issue_odd():
                issue(i, didx1, rows1, gsem1, ssem1)

            @pl.when((i >= 1) & (i <= count) & (i % 2 == 1))
            def _complete_even():
                complete(i - 1, didx0, rows0, gsem0, ssem0)

            @pl.when((i >= 1) & (i <= count) & (i % 2 == 0))
            def _complete_odd():
                complete(i - 1, didx1, rows1, gsem1, ssem1)

        pltpu.make_async_copy(rows0, accum.at[didx0], ssem0).wait()
        pltpu.make_async_copy(rows1, accum.at[didx1], ssem1).wait()
        plsc.subcore_barrier()

        @pl.when(s == 0)
        def _flush():
            pltpu.sync_copy(accum, out_hbm.at[c])

    return body(hp, src, dst)


def _dis_from(d0_ref, d1_ref):
    deg = 1.0 + d0_ref[...] + d1_ref[...]
    return lax.rsqrt(deg)


_BN = 2000


def _scaled_matmul(dp, x, w):
    n, d = x.shape
    h = w.shape[1]
    npad = dp[0].shape[0]

    def body(d0_ref, d1_ref, x_ref, w_ref, o_ref, db_ref):
        deg = 1.0 + d0_ref[...] + d1_ref[...]
        dis2 = lax.rsqrt(jnp.reshape(deg, (npad, 1)))
        dis = dis2[:n, :]
        hm = jnp.dot(x_ref[...], w_ref[...],
                     preferred_element_type=jnp.float32,
                     precision=lax.Precision.HIGHEST)
        o_ref[...] = hm * dis
        db_ref[...] = jnp.broadcast_to(dis, (n, h))

    return pl.pallas_call(
        body,
        in_specs=[
            pl.BlockSpec((npad,), lambda: (0,)),
            pl.BlockSpec((npad,), lambda: (0,)),
            pl.BlockSpec((n, d), lambda: (0, 0)),
            pl.BlockSpec((d, h), lambda: (0, 0)),
        ],
        out_specs=[pl.BlockSpec((n, h), lambda: (0, 0)),
                   pl.BlockSpec((n, h), lambda: (0, 0))],
        out_shape=[jax.ShapeDtypeStruct((n, h), jnp.float32),
                   jax.ShapeDtypeStruct((n, h), jnp.float32)],
    )(dp[0], dp[1], x, w)


def _mid_layer(aggp, hp, disb, b, w):
    n, d = hp.shape
    h = w.shape[1]

    def body(agg_ref, hp_ref, db_ref, b_ref, w_ref, o_ref):
        dis = db_ref[...]
        a = agg_ref[...]
        tot = a[0] + a[1] + hp_ref[...]
        x = jnp.maximum(tot * dis + b_ref[...], 0.0)
        hm = jnp.dot(x, w_ref[...],
                     preferred_element_type=jnp.float32,
                     precision=lax.Precision.HIGHEST)
        o_ref[...] = hm * dis

    return pl.pallas_call(
        body,
        grid=(n // _BN,),
        in_specs=[
            pl.BlockSpec((NC, _BN, d), lambda i: (0, i, 0)),
            pl.BlockSpec((_BN, d), lambda i: (i, 0)),
            pl.BlockSpec((_BN, d), lambda i: (i, 0)),
            pl.BlockSpec((1, d), lambda i: (0, 0)),
            pl.BlockSpec((d, h), lambda i: (0, 0)),
        ],
        out_specs=pl.BlockSpec((_BN, h), lambda i: (i, 0)),
        out_shape=jax.ShapeDtypeStruct((n, h), jnp.float32),
    )(aggp, hp, disb, b, w)


def _final_layer(aggp, hp, disb, b):
    n, d = hp.shape

    def body(agg_ref, hp_ref, db_ref, b_ref, o_ref):
        dis = db_ref[...]
        a = agg_ref[...]
        tot = a[0] + a[1] + hp_ref[...]
        x = jnp.maximum(tot * dis + b_ref[...], 0.0)
        m = jnp.max(x, axis=0, keepdims=True)

        @pl.when(pl.program_id(0) == 0)
        def _init():
            o_ref[...] = jnp.zeros_like(o_ref)

        o_ref[...] = jnp.maximum(o_ref[...], m)

    return pl.pallas_call(
        body,
        grid=(n // _BN,),
        in_specs=[
            pl.BlockSpec((NC, _BN, d), lambda i: (0, i, 0)),
            pl.BlockSpec((_BN, d), lambda i: (i, 0)),
            pl.BlockSpec((_BN, d), lambda i: (i, 0)),
            pl.BlockSpec((1, d), lambda i: (0, 0)),
        ],
        out_specs=pl.BlockSpec((1, d), lambda i: (0, 0)),
        out_shape=jax.ShapeDtypeStruct((1, d), jnp.float32),
    )(aggp, hp, disb, b)


def kernel(features, edge_index, edge_attr, W1, b1, W2, b2):
    n, d = features.shape
    e = edge_index.shape[1]
    assert n % _BN == 0
    dst = jnp.pad(edge_index[1], (0, 2 * _K),
                  constant_values=n)
    src = jnp.pad(lax.optimization_barrier(edge_index)[0],
                  (0, 2 * _K))
    ones_k = jnp.ones((_K,), jnp.float32)
    b1r = b1.reshape(1, -1)
    b2r = b2.reshape(1, -1)

    dp = _deg_partials(dst, ones_k, n)
    h1p, disb = _scaled_matmul(dp, features, W1)
    agg1 = _agg_partials(h1p, src, dst)
    h2p = _mid_layer(agg1, h1p, disb, b1r, W2)
    agg2 = _agg_partials(h2p, src, dst)
    return _final_layer(agg2, h2p, disb, b2r)
